# baseline (speedup 1.0000x reference)
import functools

import jax
import jax.numpy as jnp
from jax import lax
from jax.experimental import pallas as pl
from jax.experimental.pallas import tpu as pltpu
from jax.experimental.pallas import tpu_sc as plsc

VOCAB = 2
WIDTH = 1024
N_ROWS = 4 * 8192

NUM_CORES = 2
NUM_SUBCORES = 16
NUM_WORKERS = NUM_CORES * NUM_SUBCORES
ROWS_PER_WORKER = N_ROWS // NUM_WORKERS
NSEM = 4
NUM_SLICES = ROWS_PER_WORKER // 16


@functools.partial(
    pl.kernel,
    out_type=jax.ShapeDtypeStruct((N_ROWS, WIDTH), jnp.float32),
    mesh=plsc.VectorSubcoreMesh(
        core_axis_name="c", subcore_axis_name="s",
        num_cores=NUM_CORES, num_subcores=NUM_SUBCORES,
    ),
    scratch_types=[
        pltpu.VMEM((ROWS_PER_WORKER,), jnp.int32),
        pltpu.VMEM((VOCAB, WIDTH), jnp.float32),
        pltpu.VMEM((16, WIDTH), jnp.float32),
        [pltpu.SemaphoreType.DMA] * NSEM,
    ],
)
def _embed_sc(ids_hbm, table_hbm, out_hbm, idx_v, table_v, drain_v, sems):
    wid = lax.axis_index("s") * NUM_CORES + lax.axis_index("c")
    base = wid * ROWS_PER_WORKER
    pltpu.sync_copy(ids_hbm.at[pl.ds(base, ROWS_PER_WORKER)], idx_v)
    pltpu.sync_copy(table_hbm, table_v)

    def drain(sem):
        pltpu.make_async_copy(
            out_hbm.at[pl.ds(base, 16)], drain_v, sem
        ).wait()

    def issue_slice(s, sem):
        off = pl.multiple_of(s * 16, 16)
        ids16 = idx_v[pl.ds(off, 16)]
        for j in range(16):
            row_id = ids16[j]
            pltpu.async_copy(
                table_v.at[pl.ds(row_id, 1)],
                out_hbm.at[pl.ds(base + off + j, 1)],
                sem,
            )

    for b in range(NSEM):
        issue_slice(b, sems[b])

    def body(r, carry):
        for b in range(NSEM):
            drain(sems[b])
            issue_slice(r * NSEM + b, sems[b])
        return carry

    lax.fori_loop(1, NUM_SLICES // NSEM, body, 0)

    for b in range(NSEM):
        drain(sems[b])


def kernel(input, kernel):
    ids = jnp.reshape(input, (N_ROWS,)).astype(jnp.int32)
    out = _embed_sc(ids, kernel)
    return jnp.reshape(out, (4, 8192, WIDTH))
